# baseline (speedup 1.0000x reference)
import jax
import jax.numpy as jnp
from jax.experimental import pallas as pl
from jax.experimental.pallas import tpu as pltpu

_NEG_SLOPE = 0.2
_TM0 = 512
_TM1 = 1024


def _layer0_fused_kernel(adj_ref, x_ref, w0_ref, b0_ref, w1_ref, s_ref):
    t = jnp.dot(adj_ref[...].astype(jnp.bfloat16), x_ref[...],
                preferred_element_type=jnp.float32)
    h = jnp.dot(t, w0_ref[...], preferred_element_type=jnp.float32)
    h += b0_ref[...]
    h = jnp.where(h > 0, h, _NEG_SLOPE * h)
    s = jnp.dot(h, w1_ref[...], preferred_element_type=jnp.float32)
    s_ref[...] = s.astype(s_ref.dtype)


def _layer1_kernel(adj_ref, s_ref, b1_ref, o_ref):
    acc = jnp.dot(adj_ref[...].astype(jnp.bfloat16), s_ref[...],
                  preferred_element_type=jnp.float32)
    o_ref[...] = acc + b1_ref[...]


def kernel(x, adj, w0, b0, w1, b1):
    n, c0 = x.shape
    c1 = w0.shape[1]
    c2 = w1.shape[1]

    x_bf = x.astype(jnp.bfloat16)
    b0_2d = b0.reshape(1, c1)
    b1_2d = b1.reshape(1, c2)

    s = pl.pallas_call(
        _layer0_fused_kernel,
        out_shape=jax.ShapeDtypeStruct((n, c2), jnp.bfloat16),
        grid=(n // _TM0,),
        in_specs=[
            pl.BlockSpec((_TM0, n), lambda i: (i, 0)),
            pl.BlockSpec((n, c0), lambda i: (0, 0)),
            pl.BlockSpec((c0, c1), lambda i: (0, 0)),
            pl.BlockSpec((1, c1), lambda i: (0, 0)),
            pl.BlockSpec((c1, c2), lambda i: (0, 0)),
        ],
        out_specs=pl.BlockSpec((_TM0, c2), lambda i: (i, 0)),
        compiler_params=pltpu.CompilerParams(
            dimension_semantics=("parallel",)),
    )(adj, x_bf, w0, b0_2d, w1)

    out = pl.pallas_call(
        _layer1_kernel,
        out_shape=jax.ShapeDtypeStruct((n, c2), jnp.float32),
        grid=(n // _TM1,),
        in_specs=[
            pl.BlockSpec((_TM1, n), lambda i: (i, 0)),
            pl.BlockSpec((n, c2), lambda i: (0, 0)),
            pl.BlockSpec((1, c2), lambda i: (0, 0)),
        ],
        out_specs=pl.BlockSpec((_TM1, c2), lambda i: (i, 0)),
        compiler_params=pltpu.CompilerParams(
            dimension_semantics=("parallel",)),
    )(adj, s, b1_2d)

    return out


# speedup vs baseline: 1.0068x; 1.0068x over previous
import jax
import jax.numpy as jnp
from jax.experimental import pallas as pl
from jax.experimental.pallas import tpu as pltpu

_NEG_SLOPE = 0.2
_TM0 = 512
_TM1 = 512


def _layer0_fused_kernel(adj_ref, x_ref, w0_ref, b0_ref, w1_ref,
                         s_ref, adjbf_ref):
    adj_bf = adj_ref[...].astype(jnp.bfloat16)
    adjbf_ref[...] = adj_bf
    t = jnp.dot(adj_bf, x_ref[...], preferred_element_type=jnp.float32)
    h = jnp.dot(t, w0_ref[...], preferred_element_type=jnp.float32)
    h += b0_ref[...]
    h = jnp.where(h > 0, h, _NEG_SLOPE * h)
    s = jnp.dot(h, w1_ref[...], preferred_element_type=jnp.float32)
    s_ref[...] = s.astype(s_ref.dtype)


def _layer1_kernel(adj_ref, s_ref, b1_ref, o_ref):
    acc = jnp.dot(adj_ref[...], s_ref[...],
                  preferred_element_type=jnp.float32)
    o_ref[...] = acc + b1_ref[...]


def kernel(x, adj, w0, b0, w1, b1):
    n, c0 = x.shape
    c1 = w0.shape[1]
    c2 = w1.shape[1]

    x_bf = x.astype(jnp.bfloat16)
    b0_2d = b0.reshape(1, c1)
    b1_2d = b1.reshape(1, c2)

    s, adj_bf = pl.pallas_call(
        _layer0_fused_kernel,
        out_shape=(jax.ShapeDtypeStruct((n, c2), jnp.bfloat16),
                   jax.ShapeDtypeStruct((n, n), jnp.bfloat16)),
        grid=(n // _TM0,),
        in_specs=[
            pl.BlockSpec((_TM0, n), lambda i: (i, 0)),
            pl.BlockSpec((n, c0), lambda i: (0, 0)),
            pl.BlockSpec((c0, c1), lambda i: (0, 0)),
            pl.BlockSpec((1, c1), lambda i: (0, 0)),
            pl.BlockSpec((c1, c2), lambda i: (0, 0)),
        ],
        out_specs=(pl.BlockSpec((_TM0, c2), lambda i: (i, 0)),
                   pl.BlockSpec((_TM0, n), lambda i: (i, 0))),
        compiler_params=pltpu.CompilerParams(
            dimension_semantics=("parallel",)),
    )(adj, x_bf, w0, b0_2d, w1)

    out = pl.pallas_call(
        _layer1_kernel,
        out_shape=jax.ShapeDtypeStruct((n, c2), jnp.float32),
        grid=(n // _TM1,),
        in_specs=[
            pl.BlockSpec((_TM1, n), lambda i: (i, 0)),
            pl.BlockSpec((n, c2), lambda i: (0, 0)),
            pl.BlockSpec((1, c2), lambda i: (0, 0)),
        ],
        out_specs=pl.BlockSpec((_TM1, c2), lambda i: (i, 0)),
        compiler_params=pltpu.CompilerParams(
            dimension_semantics=("parallel",)),
    )(adj_bf, s, b1_2d)

    return out


# speedup vs baseline: 1.0953x; 1.0879x over previous
import jax
import jax.numpy as jnp
from jax.experimental import pallas as pl
from jax.experimental.pallas import tpu as pltpu

_NEG_SLOPE = 0.2
_TM0 = 512
_TM1 = 512


def _layer0_fused_kernel(adj_ref, x_ref, w0_ref, b0_ref, w1_ref, s_ref):
    t = jnp.dot(adj_ref[...].astype(jnp.bfloat16),
                x_ref[...].astype(jnp.bfloat16),
                preferred_element_type=jnp.float32)
    h = jnp.dot(t, w0_ref[...], preferred_element_type=jnp.float32)
    h += b0_ref[...]
    h = jnp.where(h > 0, h, _NEG_SLOPE * h)
    s = jnp.dot(h, w1_ref[...], preferred_element_type=jnp.float32)
    s_ref[...] = s.astype(s_ref.dtype)


def _layer1_kernel(adj_ref, s_ref, b1_ref, o_ref):
    acc = jnp.dot(adj_ref[...].astype(jnp.bfloat16), s_ref[...],
                  preferred_element_type=jnp.float32)
    o_ref[...] = acc + b1_ref[...]


def kernel(x, adj, w0, b0, w1, b1):
    n, c0 = x.shape
    c1 = w0.shape[1]
    c2 = w1.shape[1]

    b0_2d = b0.reshape(1, c1)
    b1_2d = b1.reshape(1, c2)

    s = pl.pallas_call(
        _layer0_fused_kernel,
        out_shape=jax.ShapeDtypeStruct((n, c2), jnp.bfloat16),
        grid=(n // _TM0,),
        in_specs=[
            pl.BlockSpec((_TM0, n), lambda i: (i, 0)),
            pl.BlockSpec((n, c0), lambda i: (0, 0)),
            pl.BlockSpec((c0, c1), lambda i: (0, 0)),
            pl.BlockSpec((1, c1), lambda i: (0, 0)),
            pl.BlockSpec((c1, c2), lambda i: (0, 0)),
        ],
        out_specs=pl.BlockSpec((_TM0, c2), lambda i: (i, 0)),
        compiler_params=pltpu.CompilerParams(
            dimension_semantics=("parallel",)),
    )(adj, x, w0, b0_2d, w1)

    out = pl.pallas_call(
        _layer1_kernel,
        out_shape=jax.ShapeDtypeStruct((n, c2), jnp.float32),
        grid=(n // _TM1,),
        in_specs=[
            pl.BlockSpec((_TM1, n), lambda i: (i, 0)),
            pl.BlockSpec((n, c2), lambda i: (0, 0)),
            pl.BlockSpec((1, c2), lambda i: (0, 0)),
        ],
        out_specs=pl.BlockSpec((_TM1, c2), lambda i: (i, 0)),
        compiler_params=pltpu.CompilerParams(
            dimension_semantics=("parallel",)),
    )(adj, s, b1_2d)

    return out
